# baseline (speedup 1.0000x reference)
import jax
import jax.numpy as jnp
from jax.experimental import pallas as pl

_TB = 32


def _loss_block_kernel(xa_ref, xb_ref, tgt_ref, reward_ref, out_ref):
    i = pl.program_id(0)
    xa = xa_ref[...]
    xb = xb_ref[...]
    tb, vh = xa.shape
    tgt = tgt_ref[0, 0, :]
    cols = jax.lax.broadcasted_iota(jnp.int32, (tb, vh), 1)
    neg = jnp.float32(-jnp.inf)
    ga = jnp.max(jnp.where(cols == tgt[:, None], xa, neg), axis=1)
    gb = jnp.max(jnp.where(cols + vh == tgt[:, None], xb, neg), axis=1)
    g = jnp.maximum(ga, gb)
    m = jnp.maximum(jnp.max(xa, axis=1), jnp.max(xb, axis=1))
    mc = m[:, None]
    s = jnp.sum(jnp.exp(xa - mc), axis=1) + jnp.sum(jnp.exp(xb - mc), axis=1)
    sel = jnp.exp(g - m) / s
    mask = (tgt > 0).astype(jnp.float32)
    partial = jnp.sum(sel * mask * reward_ref[0, 0, :])

    @pl.when(i == 0)
    def _init():
        out_ref[...] = jnp.zeros_like(out_ref)

    out_ref[...] += jnp.full(out_ref.shape, -partial, out_ref.dtype)


def kernel(preds, tgt, tgt_pos, reward):
    b, seq, v = preds.shape
    n = b * seq
    nt = n // _TB
    vh = v // 2
    preds2 = preds.reshape(n, v)
    tgt3 = tgt.reshape(nt, 1, _TB)
    reward3 = reward.reshape(nt, 1, _TB)

    out = pl.pallas_call(
        _loss_block_kernel,
        grid=(nt,),
        in_specs=[
            pl.BlockSpec((_TB, vh), lambda i: (i, 0)),
            pl.BlockSpec((_TB, vh), lambda i: (i, 1)),
            pl.BlockSpec((1, 1, _TB), lambda i: (i, 0, 0)),
            pl.BlockSpec((1, 1, _TB), lambda i: (i, 0, 0)),
        ],
        out_specs=pl.BlockSpec((1, 1), lambda i: (0, 0)),
        out_shape=jax.ShapeDtypeStruct((1, 1), jnp.float32),
    )(preds2, preds2, tgt3, reward3)
    return out[0, 0]


# speedup vs baseline: 1.3855x; 1.3855x over previous
import jax
import jax.numpy as jnp
from jax.experimental import pallas as pl

_TB = 128


def _loss_block_kernel(xa_ref, xb_ref, tgt_ref, reward_ref, out_ref):
    i = pl.program_id(0)
    xa = xa_ref[...]
    xb = xb_ref[...]
    tb, vh = xa.shape
    tgt = tgt_ref[0, 0, :]
    cols = jax.lax.broadcasted_iota(jnp.int32, (tb, vh), 1)
    neg = jnp.float32(-jnp.inf)
    ga = jnp.max(jnp.where(cols == tgt[:, None], xa, neg), axis=1)
    gb = jnp.max(jnp.where(cols + vh == tgt[:, None], xb, neg), axis=1)
    g = jnp.maximum(ga, gb)
    m = jnp.maximum(jnp.max(xa, axis=1), jnp.max(xb, axis=1))
    mc = m[:, None]
    s = jnp.sum(jnp.exp(xa - mc), axis=1) + jnp.sum(jnp.exp(xb - mc), axis=1)
    sel = jnp.exp(g - m) / s
    mask = (tgt > 0).astype(jnp.float32)
    partial = jnp.sum(sel * mask * reward_ref[0, 0, :])

    @pl.when(i == 0)
    def _init():
        out_ref[...] = jnp.zeros_like(out_ref)

    out_ref[...] += jnp.full(out_ref.shape, -partial, out_ref.dtype)


def kernel(preds, tgt, tgt_pos, reward):
    b, seq, v = preds.shape
    n = b * seq
    nt = n // _TB
    vh = v // 2
    preds2 = preds.reshape(n, v)
    tgt3 = tgt.reshape(nt, 1, _TB)
    reward3 = reward.reshape(nt, 1, _TB)

    out = pl.pallas_call(
        _loss_block_kernel,
        grid=(nt,),
        in_specs=[
            pl.BlockSpec((_TB, vh), lambda i: (i, 0)),
            pl.BlockSpec((_TB, vh), lambda i: (i, 1)),
            pl.BlockSpec((1, 1, _TB), lambda i: (i, 0, 0)),
            pl.BlockSpec((1, 1, _TB), lambda i: (i, 0, 0)),
        ],
        out_specs=pl.BlockSpec((1, 1), lambda i: (0, 0)),
        out_shape=jax.ShapeDtypeStruct((1, 1), jnp.float32),
    )(preds2, preds2, tgt3, reward3)
    return out[0, 0]


# speedup vs baseline: 1.4403x; 1.0396x over previous
import jax
import jax.numpy as jnp
from jax.experimental import pallas as pl
from jax.experimental.pallas import tpu as pltpu

_TB = 256


def _loss_block_kernel(xa_ref, xb_ref, tgt_ref, reward_ref, out_ref):
    i = pl.program_id(0)
    xa = xa_ref[...]
    xb = xb_ref[...]
    tb, vh = xa.shape
    tgt = tgt_ref[0, 0, :]
    cols = jax.lax.broadcasted_iota(jnp.int32, (tb, vh), 1)
    neg = jnp.float32(-jnp.inf)
    ga = jnp.max(jnp.where(cols == tgt[:, None], xa, neg), axis=1)
    gb = jnp.max(jnp.where(cols + vh == tgt[:, None], xb, neg), axis=1)
    g = jnp.maximum(ga, gb)
    m = jnp.maximum(jnp.max(xa, axis=1), jnp.max(xb, axis=1))
    mc = m[:, None]
    s = jnp.sum(jnp.exp(xa - mc), axis=1) + jnp.sum(jnp.exp(xb - mc), axis=1)
    sel = jnp.exp(g - m) / s
    mask = (tgt > 0).astype(jnp.float32)
    partial = jnp.sum(sel * mask * reward_ref[0, 0, :])

    @pl.when(i == 0)
    def _init():
        out_ref[...] = jnp.zeros_like(out_ref)

    out_ref[...] += jnp.full(out_ref.shape, -partial, out_ref.dtype)


def kernel(preds, tgt, tgt_pos, reward):
    b, seq, v = preds.shape
    n = b * seq
    nt = n // _TB
    vh = v // 2
    preds2 = preds.reshape(n, v)
    tgt3 = tgt.reshape(nt, 1, _TB)
    reward3 = reward.reshape(nt, 1, _TB)

    out = pl.pallas_call(
        _loss_block_kernel,
        grid=(nt,),
        in_specs=[
            pl.BlockSpec((_TB, vh), lambda i: (i, 0)),
            pl.BlockSpec((_TB, vh), lambda i: (i, 1)),
            pl.BlockSpec((1, 1, _TB), lambda i: (i, 0, 0)),
            pl.BlockSpec((1, 1, _TB), lambda i: (i, 0, 0)),
        ],
        out_specs=pl.BlockSpec((1, 1), lambda i: (0, 0)),
        out_shape=jax.ShapeDtypeStruct((1, 1), jnp.float32),
        compiler_params=pltpu.CompilerParams(
            vmem_limit_bytes=110 * 1024 * 1024,
        ),
    )(preds2, preds2, tgt3, reward3)
    return out[0, 0]
